# baseline (speedup 1.0000x reference)
import functools

import jax
import jax.numpy as jnp
from jax import lax
from jax.experimental import pallas as pl
from jax.experimental.pallas import tpu as pltpu
from jax.experimental.pallas import tpu_sc as plsc

N_NODES = 10000
N_EDGES = 160000
D = 128
EH = N_EDGES // 2

NC = 2
NS = 16
NW = NC * NS
CHUNK = 128
NCH = EH // CHUNK
NCHP = ((NCH + NW - 1) // NW) * NW
EHP = NCHP * CHUNK
CPW = NCHP // NW
GG = 2
SGC = 8
SGPW = 2 * (NCHP // SGC) // NW
ROWS_PER_TILE = 624
ROWS_TAIL = N_NODES - NS * ROWS_PER_TILE

_mesh = plsc.VectorSubcoreMesh(core_axis_name="c", subcore_axis_name="s")


def _gather_body(ab_hbm, cidx_hbm, og_hbm, cbuf, rows, sem):
    c = lax.axis_index("c")
    s = lax.axis_index("s")
    wid = s * NC + c
    c0 = wid * CPW
    for g in range(CPW // GG):
        base = (c0 + GG * g) * 2 * CHUNK
        pltpu.sync_copy(cidx_hbm.at[pl.ds(base, 2 * GG * CHUNK)], cbuf)
        cps = [
            pltpu.async_copy(
                ab_hbm.at[cbuf.at[pl.ds(j * CHUNK, CHUNK)]],
                rows.at[pl.ds(j * CHUNK, CHUNK)], sem)
            for j in range(2 * GG)
        ]
        for cp in cps:
            cp.wait()
        pltpu.sync_copy(rows, og_hbm.at[pl.ds(base, 2 * GG * CHUNK)])


_gather_half = functools.partial(
    pl.kernel,
    out_type=jax.ShapeDtypeStruct((2 * EHP, D), jnp.float32),
    mesh=_mesh,
    scratch_types=[
        pltpu.VMEM((2 * GG * CHUNK,), jnp.int32),
        pltpu.VMEM((2 * GG * CHUNK, D), jnp.float32),
        pltpu.SemaphoreType.DMA,
    ],
)(_gather_body)


def _scatter_body(ea_hbm, eb_hbm, dsta_hbm, dstb_hbm, zero_hbm, out_hbm,
                  didx, rows, shared):
    c = lax.axis_index("c")
    s = lax.axis_index("s")
    wid = s * NC + c
    row0 = s * ROWS_PER_TILE
    tail0 = NS * ROWS_PER_TILE
    pltpu.sync_copy(zero_hbm.at[pl.ds(row0, ROWS_PER_TILE)],
                    shared.at[pl.ds(row0, ROWS_PER_TILE)])

    @pl.when(s == NS - 1)
    def _():
        pltpu.sync_copy(zero_hbm.at[pl.ds(tail0, ROWS_TAIL)],
                        shared.at[pl.ds(tail0, ROWS_TAIL)])

    plsc.subcore_barrier()

    gph = NCHP // SGC
    for gl in range(SGPW):
        gg = wid * SGPW + gl
        half = gg // gph
        g = gg - half * gph
        in_a = half == 0

        @pl.when(in_a)
        def _():
            pltpu.sync_copy(dsta_hbm.at[pl.ds(g * SGC, SGC)], didx)

        @pl.when(jnp.logical_not(in_a))
        def _():
            pltpu.sync_copy(dstb_hbm.at[pl.ds(g * SGC, SGC)], didx)

        for sub in range(4):
            ebase = g * SGC * CHUNK + sub * 2 * CHUNK

            @pl.when(in_a)
            def _():
                pltpu.sync_copy(ea_hbm.at[pl.ds(ebase, 2 * CHUNK)], rows)

            @pl.when(jnp.logical_not(in_a))
            def _():
                pltpu.sync_copy(eb_hbm.at[pl.ds(ebase, 2 * CHUNK)], rows)

            for j in range(2):
                pltpu.sync_copy(rows.at[pl.ds(j * CHUNK, CHUNK)],
                                shared.at[didx.at[sub * 2 + j]], add=True)

    plsc.subcore_barrier()
    pltpu.sync_copy(shared.at[pl.ds(row0, ROWS_PER_TILE)],
                    out_hbm.at[c, pl.ds(row0, ROWS_PER_TILE)])

    @pl.when(s == NS - 1)
    def _():
        pltpu.sync_copy(shared.at[pl.ds(tail0, ROWS_TAIL)],
                        out_hbm.at[c, pl.ds(tail0, ROWS_TAIL)])


_scatter_call = functools.partial(
    pl.kernel,
    out_type=jax.ShapeDtypeStruct((NC, N_NODES, D), jnp.float32),
    mesh=_mesh,
    scratch_types=[
        pltpu.VMEM((SGC, CHUNK), jnp.int32),
        pltpu.VMEM((2 * CHUNK, D), jnp.float32),
        pltpu.VMEM_SHARED((N_NODES, D), jnp.float32),
    ],
)(_scatter_body)


BEC = 8
BE = BEC * CHUNK
BN = 2000


def _projab_body(hn, w, out):
    out[...] = jnp.dot(hn[...], w[0],
                       preferred_element_type=jnp.float32)


def _edge_body(og, he, w1, b1, w2, b2, g, bb, out):
    he_v = he[...]
    x2 = og[...].reshape(BEC, 2, CHUNK, D)
    g12 = (x2[:, 0] + x2[:, 1]).reshape(BE, D)
    x = g12 + b1[...] + jnp.dot(he_v, w1[...],
                                preferred_element_type=jnp.float32)
    h = jnp.maximum(x, 0.0)
    y = jnp.dot(h, w2[...], preferred_element_type=jnp.float32) + b2[...]
    mu = jnp.mean(y, axis=-1, keepdims=True)
    yc = y - mu
    var = jnp.mean(yc * yc, axis=-1, keepdims=True)
    res = he_v + yc * lax.rsqrt(var + 1e-5) * g[...] + bb[...]
    i = pl.program_id(0)
    rid = i * BE + lax.broadcasted_iota(jnp.int32, (BE, 1), 0)
    out[...] = jnp.where(rid < EH, res, 0.0)


def _node_body(hn, p0, p1, w1a, w1b, b1, w2, b2, g, bb, out_h):
    hn_v = hn[...]
    agg = p0[...] + p1[...]
    x = (jnp.dot(hn_v, w1a[...], preferred_element_type=jnp.float32)
         + jnp.dot(agg, w1b[...], preferred_element_type=jnp.float32)
         + b1[...])
    h = jnp.maximum(x, 0.0)
    y = jnp.dot(h, w2[...], preferred_element_type=jnp.float32) + b2[...]
    mu = jnp.mean(y, axis=-1, keepdims=True)
    yc = y - mu
    var = jnp.mean(yc * yc, axis=-1, keepdims=True)
    out_h[...] = hn_v + yc * lax.rsqrt(var + 1e-5) * g[...] + bb[...]


def _row_spec(bs):
    return pl.BlockSpec((bs, D), lambda i: (i, 0))


def _w_spec():
    return pl.BlockSpec((D, D), lambda i: (0, 0))


def _b_spec():
    return pl.BlockSpec((1, D), lambda i: (0, 0))


_projab_call = pl.pallas_call(
    _projab_body,
    grid=(2, N_NODES // BN),
    in_specs=[pl.BlockSpec((BN, D), lambda j, i: (i, 0)),
              pl.BlockSpec((1, D, D), lambda j, i: (j, 0, 0))],
    out_specs=pl.BlockSpec((BN, D), lambda j, i: (j * (N_NODES // BN) + i, 0)),
    out_shape=jax.ShapeDtypeStruct((2 * N_NODES, D), jnp.float32),
)

_edge_call = pl.pallas_call(
    _edge_body,
    grid=(EHP // BE,),
    in_specs=[pl.BlockSpec((2 * BE, D), lambda i: (i, 0)), _row_spec(BE),
              _w_spec(), _b_spec(), _w_spec(), _b_spec(),
              _b_spec(), _b_spec()],
    out_specs=_row_spec(BE),
    out_shape=jax.ShapeDtypeStruct((EHP, D), jnp.float32),
)

_node_call = pl.pallas_call(
    _node_body,
    grid=(N_NODES // BN,),
    in_specs=[_row_spec(BN), _row_spec(BN), _row_spec(BN),
              _w_spec(), _w_spec(), _b_spec(), _w_spec(), _b_spec(),
              _b_spec(), _b_spec()],
    out_specs=_row_spec(BN),
    out_shape=jax.ShapeDtypeStruct((N_NODES, D), jnp.float32),
)


def _pad_idx(idx):
    return jnp.concatenate([idx, jnp.zeros((EHP - EH,), jnp.int32)])


def _interleave(src, dst):
    s2 = _pad_idx(src).reshape(NCHP, CHUNK)
    d2 = _pad_idx(dst).reshape(NCHP, CHUNK) + N_NODES
    return jnp.stack([s2, d2], axis=1).reshape(2 * EHP)


def kernel(h_node, h_edge, edge_index, We1, be1, We2, be2, ge, bbe,
           Wn1, bn1, Wn2, bn2, gn, bbn):
    src_a, src_b = edge_index[0, :EH], edge_index[0, EH:]
    dst_a, dst_b = edge_index[1, :EH], edge_index[1, EH:]
    cidx_a = _interleave(src_a, dst_a)
    cidx_b = _interleave(src_b, dst_b)
    dsta2d = _pad_idx(dst_a).reshape(NCHP, CHUNK)
    dstb2d = _pad_idx(dst_b).reshape(NCHP, CHUNK)
    pad = jnp.zeros((EHP - EH, D), jnp.float32)
    he_a = jnp.concatenate([h_edge[:EH], pad])
    he_b = jnp.concatenate([h_edge[EH:], pad])
    zeros = jnp.zeros((N_NODES, D), jnp.float32)
    num_convs = We1.shape[0]
    We1_ab = jnp.stack([We1[:, :D], We1[:, D:2 * D]], axis=1)

    ab = _projab_call(h_node, We1_ab[0])
    for i in range(num_convs):
        ew = (We1[i, 2 * D:], be1[i].reshape(1, D), We2[i],
              be2[i].reshape(1, D), ge[i].reshape(1, D), bbe[i].reshape(1, D))
        og_a = _gather_half(ab, cidx_a)
        og_b = _gather_half(ab, cidx_b)
        he_a = _edge_call(og_a, he_a, *ew)
        he_b = _edge_call(og_b, he_b, *ew)
        partials = _scatter_call(he_a, he_b, dsta2d, dstb2d, zeros)
        h_node = _node_call(
            h_node, partials[0], partials[1],
            Wn1[i, :D], Wn1[i, D:], bn1[i].reshape(1, D),
            Wn2[i], bn2[i].reshape(1, D),
            gn[i].reshape(1, D), bbn[i].reshape(1, D))
        if i + 1 < num_convs:
            ab = _projab_call(h_node, We1_ab[i + 1])
    return h_node, jnp.concatenate([he_a[:EH], he_b[:EH]], axis=0)


# speedup vs baseline: 1.0036x; 1.0036x over previous
import functools

import jax
import jax.numpy as jnp
from jax import lax
from jax.experimental import pallas as pl
from jax.experimental.pallas import tpu as pltpu
from jax.experimental.pallas import tpu_sc as plsc

N_NODES = 10000
N_EDGES = 160000
D = 128
EH = N_EDGES // 2

NC = 2
NS = 16
NW = NC * NS
CHUNK = 128
NCH = EH // CHUNK
NCHP = ((NCH + NW - 1) // NW) * NW
EHP = NCHP * CHUNK
CPW = NCHP // NW
GG = 2
SGC = 8
SGPW = 2 * (NCHP // SGC) // NW
ROWS_PER_TILE = 624
ROWS_TAIL = N_NODES - NS * ROWS_PER_TILE

_mesh = plsc.VectorSubcoreMesh(core_axis_name="c", subcore_axis_name="s")


def _gather_body(ab_hbm, cidx_hbm, og_hbm, cbuf, rows, sem):
    c = lax.axis_index("c")
    s = lax.axis_index("s")
    wid = s * NC + c
    c0 = wid * CPW

    def step(g, carry):
        base = (c0 + GG * g) * 2 * CHUNK
        pltpu.sync_copy(cidx_hbm.at[pl.ds(base, 2 * GG * CHUNK)], cbuf)
        cps = [
            pltpu.async_copy(
                ab_hbm.at[cbuf.at[pl.ds(j * CHUNK, CHUNK)]],
                rows.at[pl.ds(j * CHUNK, CHUNK)], sem)
            for j in range(2 * GG)
        ]
        for cp in cps:
            cp.wait()
        pltpu.sync_copy(rows, og_hbm.at[pl.ds(base, 2 * GG * CHUNK)])
        return carry

    lax.fori_loop(0, CPW // GG, step, 0)


_gather_half = functools.partial(
    pl.kernel,
    out_type=jax.ShapeDtypeStruct((2 * EHP, D), jnp.float32),
    mesh=_mesh,
    scratch_types=[
        pltpu.VMEM((2 * GG * CHUNK,), jnp.int32),
        pltpu.VMEM((2 * GG * CHUNK, D), jnp.float32),
        pltpu.SemaphoreType.DMA,
    ],
)(_gather_body)


def _scatter_body(ea_hbm, eb_hbm, dsta_hbm, dstb_hbm, zero_hbm, out_hbm,
                  didx, rows, shared):
    c = lax.axis_index("c")
    s = lax.axis_index("s")
    wid = s * NC + c
    row0 = s * ROWS_PER_TILE
    tail0 = NS * ROWS_PER_TILE
    pltpu.sync_copy(zero_hbm.at[pl.ds(row0, ROWS_PER_TILE)],
                    shared.at[pl.ds(row0, ROWS_PER_TILE)])

    @pl.when(s == NS - 1)
    def _():
        pltpu.sync_copy(zero_hbm.at[pl.ds(tail0, ROWS_TAIL)],
                        shared.at[pl.ds(tail0, ROWS_TAIL)])

    plsc.subcore_barrier()

    gph = NCHP // SGC

    def step(gl, carry):
        gg = wid * SGPW + gl
        half = gg // gph
        g = gg - half * gph
        in_a = half == 0

        @pl.when(in_a)
        def _():
            pltpu.sync_copy(dsta_hbm.at[pl.ds(g * SGC, SGC)], didx)

        @pl.when(jnp.logical_not(in_a))
        def _():
            pltpu.sync_copy(dstb_hbm.at[pl.ds(g * SGC, SGC)], didx)

        for sub in range(4):
            ebase = g * SGC * CHUNK + sub * 2 * CHUNK

            @pl.when(in_a)
            def _():
                pltpu.sync_copy(ea_hbm.at[pl.ds(ebase, 2 * CHUNK)], rows)

            @pl.when(jnp.logical_not(in_a))
            def _():
                pltpu.sync_copy(eb_hbm.at[pl.ds(ebase, 2 * CHUNK)], rows)

            for j in range(2):
                pltpu.sync_copy(rows.at[pl.ds(j * CHUNK, CHUNK)],
                                shared.at[didx.at[sub * 2 + j]], add=True)
        return carry

    lax.fori_loop(0, SGPW, step, 0)

    plsc.subcore_barrier()
    pltpu.sync_copy(shared.at[pl.ds(row0, ROWS_PER_TILE)],
                    out_hbm.at[c, pl.ds(row0, ROWS_PER_TILE)])

    @pl.when(s == NS - 1)
    def _():
        pltpu.sync_copy(shared.at[pl.ds(tail0, ROWS_TAIL)],
                        out_hbm.at[c, pl.ds(tail0, ROWS_TAIL)])


_scatter_call = functools.partial(
    pl.kernel,
    out_type=jax.ShapeDtypeStruct((NC, N_NODES, D), jnp.float32),
    mesh=_mesh,
    scratch_types=[
        pltpu.VMEM((SGC, CHUNK), jnp.int32),
        pltpu.VMEM((2 * CHUNK, D), jnp.float32),
        pltpu.VMEM_SHARED((N_NODES, D), jnp.float32),
    ],
)(_scatter_body)


BEC = 8
BE = BEC * CHUNK
BN = 2000


def _projab_body(hn, w, out):
    out[...] = jnp.dot(hn[...], w[0],
                       preferred_element_type=jnp.float32)


def _edge_body(og, he, w1, b1, w2, b2, g, bb, out):
    he_v = he[...]
    x2 = og[...].reshape(BEC, 2, CHUNK, D)
    g12 = (x2[:, 0] + x2[:, 1]).reshape(BE, D)
    x = g12 + b1[...] + jnp.dot(he_v, w1[...],
                                preferred_element_type=jnp.float32)
    h = jnp.maximum(x, 0.0)
    y = jnp.dot(h, w2[...], preferred_element_type=jnp.float32) + b2[...]
    mu = jnp.mean(y, axis=-1, keepdims=True)
    yc = y - mu
    var = jnp.mean(yc * yc, axis=-1, keepdims=True)
    res = he_v + yc * lax.rsqrt(var + 1e-5) * g[...] + bb[...]
    i = pl.program_id(0)
    rid = i * BE + lax.broadcasted_iota(jnp.int32, (BE, 1), 0)
    out[...] = jnp.where(rid < EH, res, 0.0)


def _node_body(hn, p0, p1, w1a, w1b, b1, w2, b2, g, bb, out_h):
    hn_v = hn[...]
    agg = p0[...] + p1[...]
    x = (jnp.dot(hn_v, w1a[...], preferred_element_type=jnp.float32)
         + jnp.dot(agg, w1b[...], preferred_element_type=jnp.float32)
         + b1[...])
    h = jnp.maximum(x, 0.0)
    y = jnp.dot(h, w2[...], preferred_element_type=jnp.float32) + b2[...]
    mu = jnp.mean(y, axis=-1, keepdims=True)
    yc = y - mu
    var = jnp.mean(yc * yc, axis=-1, keepdims=True)
    out_h[...] = hn_v + yc * lax.rsqrt(var + 1e-5) * g[...] + bb[...]


def _row_spec(bs):
    return pl.BlockSpec((bs, D), lambda i: (i, 0))


def _w_spec():
    return pl.BlockSpec((D, D), lambda i: (0, 0))


def _b_spec():
    return pl.BlockSpec((1, D), lambda i: (0, 0))


_projab_call = pl.pallas_call(
    _projab_body,
    grid=(2, N_NODES // BN),
    in_specs=[pl.BlockSpec((BN, D), lambda j, i: (i, 0)),
              pl.BlockSpec((1, D, D), lambda j, i: (j, 0, 0))],
    out_specs=pl.BlockSpec((BN, D), lambda j, i: (j * (N_NODES // BN) + i, 0)),
    out_shape=jax.ShapeDtypeStruct((2 * N_NODES, D), jnp.float32),
)

_edge_call = pl.pallas_call(
    _edge_body,
    grid=(EHP // BE,),
    in_specs=[pl.BlockSpec((2 * BE, D), lambda i: (i, 0)), _row_spec(BE),
              _w_spec(), _b_spec(), _w_spec(), _b_spec(),
              _b_spec(), _b_spec()],
    out_specs=_row_spec(BE),
    out_shape=jax.ShapeDtypeStruct((EHP, D), jnp.float32),
)

_node_call = pl.pallas_call(
    _node_body,
    grid=(N_NODES // BN,),
    in_specs=[_row_spec(BN), _row_spec(BN), _row_spec(BN),
              _w_spec(), _w_spec(), _b_spec(), _w_spec(), _b_spec(),
              _b_spec(), _b_spec()],
    out_specs=_row_spec(BN),
    out_shape=jax.ShapeDtypeStruct((N_NODES, D), jnp.float32),
)


def _pad_idx(idx):
    return jnp.concatenate([idx, jnp.zeros((EHP - EH,), jnp.int32)])


def _interleave(src, dst):
    s2 = _pad_idx(src).reshape(NCHP, CHUNK)
    d2 = _pad_idx(dst).reshape(NCHP, CHUNK) + N_NODES
    return jnp.stack([s2, d2], axis=1).reshape(2 * EHP)


def kernel(h_node, h_edge, edge_index, We1, be1, We2, be2, ge, bbe,
           Wn1, bn1, Wn2, bn2, gn, bbn):
    src_a, src_b = edge_index[0, :EH], edge_index[0, EH:]
    dst_a, dst_b = edge_index[1, :EH], edge_index[1, EH:]
    cidx_a = _interleave(src_a, dst_a)
    cidx_b = _interleave(src_b, dst_b)
    dsta2d = _pad_idx(dst_a).reshape(NCHP, CHUNK)
    dstb2d = _pad_idx(dst_b).reshape(NCHP, CHUNK)
    pad = jnp.zeros((EHP - EH, D), jnp.float32)
    he_a = jnp.concatenate([h_edge[:EH], pad])
    he_b = jnp.concatenate([h_edge[EH:], pad])
    zeros = jnp.zeros((N_NODES, D), jnp.float32)
    num_convs = We1.shape[0]
    We1_ab = jnp.stack([We1[:, :D], We1[:, D:2 * D]], axis=1)

    ab = _projab_call(h_node, We1_ab[0])
    for i in range(num_convs):
        ew = (We1[i, 2 * D:], be1[i].reshape(1, D), We2[i],
              be2[i].reshape(1, D), ge[i].reshape(1, D), bbe[i].reshape(1, D))
        og_a = _gather_half(ab, cidx_a)
        og_b = _gather_half(ab, cidx_b)
        he_a = _edge_call(og_a, he_a, *ew)
        he_b = _edge_call(og_b, he_b, *ew)
        partials = _scatter_call(he_a, he_b, dsta2d, dstb2d, zeros)
        h_node = _node_call(
            h_node, partials[0], partials[1],
            Wn1[i, :D], Wn1[i, D:], bn1[i].reshape(1, D),
            Wn2[i], bn2[i].reshape(1, D),
            gn[i].reshape(1, D), bbn[i].reshape(1, D))
        if i + 1 < num_convs:
            ab = _projab_call(h_node, We1_ab[i + 1])
    return h_node, jnp.concatenate([he_a[:EH], he_b[:EH]], axis=0)


# speedup vs baseline: 1.2207x; 1.2163x over previous
import functools

import jax
import jax.numpy as jnp
from jax import lax
from jax.experimental import pallas as pl
from jax.experimental.pallas import tpu as pltpu
from jax.experimental.pallas import tpu_sc as plsc

N_NODES = 10000
N_EDGES = 160000
D = 128
EH = N_EDGES // 2

NC = 2
NS = 16
NW = NC * NS
CHUNK = 128
NCH = EH // CHUNK
NCHP = ((NCH + NW - 1) // NW) * NW
EHP = NCHP * CHUNK
CPW = NCHP // NW
NBUF = 3
ROWS_PER_TILE = 624
ROWS_TAIL = N_NODES - NS * ROWS_PER_TILE

_mesh = plsc.VectorSubcoreMesh(core_axis_name="c", subcore_axis_name="s")


def _gather_body(a_hbm, b_hbm, src_hbm, dst_hbm, o1_hbm, o2_hbm,
                 sidx, didx, r_a, r_b, *sems):
    sga = sems[0:NBUF]
    sgb = sems[NBUF:2 * NBUF]
    swa = sems[2 * NBUF:3 * NBUF]
    swb = sems[3 * NBUF:4 * NBUF]
    c = lax.axis_index("c")
    s = lax.axis_index("s")
    wid = s * NC + c

    def cbase(k):
        return (wid + k * NW) * CHUNK

    def start_chunk(k, b):
        base = cbase(k)
        pltpu.sync_copy(src_hbm.at[pl.ds(base, CHUNK)], sidx.at[b])
        pltpu.sync_copy(dst_hbm.at[pl.ds(base, CHUNK)], didx.at[b])
        pltpu.async_copy(a_hbm.at[sidx.at[b]], r_a.at[b], sga[b])
        pltpu.async_copy(b_hbm.at[didx.at[b]], r_b.at[b], sgb[b])

    def wait_gather(b):
        pltpu.make_async_copy(a_hbm.at[pl.ds(0, CHUNK)], r_a.at[b],
                              sga[b]).wait()
        pltpu.make_async_copy(b_hbm.at[pl.ds(0, CHUNK)], r_b.at[b],
                              sgb[b]).wait()

    def start_wb(k, b):
        base = cbase(k)
        pltpu.async_copy(r_a.at[b], o1_hbm.at[pl.ds(base, CHUNK)], swa[b])
        pltpu.async_copy(r_b.at[b], o2_hbm.at[pl.ds(base, CHUNK)], swb[b])

    def wait_wb(b):
        pltpu.make_async_copy(r_a.at[b], o1_hbm.at[pl.ds(0, CHUNK)],
                              swa[b]).wait()
        pltpu.make_async_copy(r_b.at[b], o2_hbm.at[pl.ds(0, CHUNK)],
                              swb[b]).wait()

    for p in range(NBUF):
        start_chunk(p, p)
        if p >= 1:
            wait_gather(p - 1)
            start_wb(p - 1, p - 1)

    def loop_body(g, carry):
        for b in range(NBUF):
            k = NBUF * g + b
            bp = (b + NBUF - 1) % NBUF

            @pl.when(k <= CPW - 1)
            def _():
                wait_wb(b)
                start_chunk(k, b)

            @pl.when(k <= CPW)
            def _():
                wait_gather(bp)
                start_wb(k - 1, bp)

        return carry

    lax.fori_loop(1, CPW // NBUF + 1, loop_body, 0)

    for k in range(CPW - NBUF, CPW):
        wait_wb(k % NBUF)


_gather_half = functools.partial(
    pl.kernel,
    out_type=[jax.ShapeDtypeStruct((EHP, D), jnp.float32),
              jax.ShapeDtypeStruct((EHP, D), jnp.float32)],
    mesh=_mesh,
    scratch_types=[
        pltpu.VMEM((NBUF, CHUNK), jnp.int32),
        pltpu.VMEM((NBUF, CHUNK), jnp.int32),
        pltpu.VMEM((NBUF, CHUNK, D), jnp.float32),
        pltpu.VMEM((NBUF, CHUNK, D), jnp.float32),
    ] + [pltpu.SemaphoreType.DMA] * (4 * NBUF),
)(_gather_body)


def _scatter_body(ea_hbm, eb_hbm, dsta_hbm, dstb_hbm, zero_hbm, out_hbm,
                  didx, rows_e, shared):
    c = lax.axis_index("c")
    s = lax.axis_index("s")
    wid = s * NC + c
    row0 = s * ROWS_PER_TILE
    tail0 = NS * ROWS_PER_TILE
    pltpu.sync_copy(zero_hbm.at[pl.ds(row0, ROWS_PER_TILE)],
                    shared.at[pl.ds(row0, ROWS_PER_TILE)])

    @pl.when(s == NS - 1)
    def _():
        pltpu.sync_copy(zero_hbm.at[pl.ds(tail0, ROWS_TAIL)],
                        shared.at[pl.ds(tail0, ROWS_TAIL)])

    plsc.subcore_barrier()

    nchunks = EH // CHUNK
    n_my = (nchunks - wid + NW - 1) // NW
    for e_hbm, dst_hbm in ((ea_hbm, dsta_hbm), (eb_hbm, dstb_hbm)):
        def step(k, carry, e_hbm=e_hbm, dst_hbm=dst_hbm):
            base = (wid + k * NW) * CHUNK
            pltpu.sync_copy(dst_hbm.at[pl.ds(base, CHUNK)], didx)
            pltpu.sync_copy(e_hbm.at[pl.ds(base, CHUNK)], rows_e)
            pltpu.sync_copy(rows_e, shared.at[didx], add=True)
            return carry

        lax.fori_loop(0, n_my, step, 0)

    plsc.subcore_barrier()
    pltpu.sync_copy(shared.at[pl.ds(row0, ROWS_PER_TILE)],
                    out_hbm.at[c, pl.ds(row0, ROWS_PER_TILE)])

    @pl.when(s == NS - 1)
    def _():
        pltpu.sync_copy(shared.at[pl.ds(tail0, ROWS_TAIL)],
                        out_hbm.at[c, pl.ds(tail0, ROWS_TAIL)])


_scatter_call = functools.partial(
    pl.kernel,
    out_type=jax.ShapeDtypeStruct((NC, N_NODES, D), jnp.float32),
    mesh=_mesh,
    scratch_types=[
        pltpu.VMEM((CHUNK,), jnp.int32),
        pltpu.VMEM((CHUNK, D), jnp.float32),
        pltpu.VMEM_SHARED((N_NODES, D), jnp.float32),
    ],
)(_scatter_body)


BE = 2000
BN = 2000


def _proj_body(hn, ws, wd, out_a, out_b):
    x = hn[...]
    out_a[...] = jnp.dot(x, ws[...], preferred_element_type=jnp.float32)
    out_b[...] = jnp.dot(x, wd[...], preferred_element_type=jnp.float32)


def _edge_body(g1, g2, he, w1, b1, w2, b2, g, bb, out):
    he_v = he[...]
    x = g1[...] + g2[...] + b1[...] + jnp.dot(
        he_v, w1[...], preferred_element_type=jnp.float32)
    h = jnp.maximum(x, 0.0)
    y = jnp.dot(h, w2[...], preferred_element_type=jnp.float32) + b2[...]
    mu = jnp.mean(y, axis=-1, keepdims=True)
    yc = y - mu
    var = jnp.mean(yc * yc, axis=-1, keepdims=True)
    out[...] = he_v + yc * lax.rsqrt(var + 1e-5) * g[...] + bb[...]


def _node_body(hn, p0, p1, w1a, w1b, b1, w2, b2, g, bb, ws, wd,
               out_h, out_a, out_b):
    hn_v = hn[...]
    agg = p0[...] + p1[...]
    x = (jnp.dot(hn_v, w1a[...], preferred_element_type=jnp.float32)
         + jnp.dot(agg, w1b[...], preferred_element_type=jnp.float32)
         + b1[...])
    h = jnp.maximum(x, 0.0)
    y = jnp.dot(h, w2[...], preferred_element_type=jnp.float32) + b2[...]
    mu = jnp.mean(y, axis=-1, keepdims=True)
    yc = y - mu
    var = jnp.mean(yc * yc, axis=-1, keepdims=True)
    hn_new = hn_v + yc * lax.rsqrt(var + 1e-5) * g[...] + bb[...]
    out_h[...] = hn_new
    out_a[...] = jnp.dot(hn_new, ws[...], preferred_element_type=jnp.float32)
    out_b[...] = jnp.dot(hn_new, wd[...], preferred_element_type=jnp.float32)


def _row_spec(bs):
    return pl.BlockSpec((bs, D), lambda i: (i, 0))


def _w_spec():
    return pl.BlockSpec((D, D), lambda i: (0, 0))


def _b_spec():
    return pl.BlockSpec((1, D), lambda i: (0, 0))


_proj_call = pl.pallas_call(
    _proj_body,
    grid=(N_NODES // BN,),
    in_specs=[_row_spec(BN), _w_spec(), _w_spec()],
    out_specs=[_row_spec(BN), _row_spec(BN)],
    out_shape=[jax.ShapeDtypeStruct((N_NODES, D), jnp.float32)] * 2,
)

_edge_call = pl.pallas_call(
    _edge_body,
    grid=(EH // BE,),
    in_specs=[_row_spec(BE), _row_spec(BE), _row_spec(BE),
              _w_spec(), _b_spec(), _w_spec(), _b_spec(),
              _b_spec(), _b_spec()],
    out_specs=_row_spec(BE),
    out_shape=jax.ShapeDtypeStruct((EH, D), jnp.float32),
)

_node_call = pl.pallas_call(
    _node_body,
    grid=(N_NODES // BN,),
    in_specs=[_row_spec(BN), _row_spec(BN), _row_spec(BN),
              _w_spec(), _w_spec(), _b_spec(), _w_spec(), _b_spec(),
              _b_spec(), _b_spec(), _w_spec(), _w_spec()],
    out_specs=[_row_spec(BN), _row_spec(BN), _row_spec(BN)],
    out_shape=[jax.ShapeDtypeStruct((N_NODES, D), jnp.float32)] * 3,
)


def _pad_idx(idx):
    return jnp.concatenate([idx, jnp.zeros((EHP - EH,), jnp.int32)])


def kernel(h_node, h_edge, edge_index, We1, be1, We2, be2, ge, bbe,
           Wn1, bn1, Wn2, bn2, gn, bbn):
    src_a, src_b = edge_index[0, :EH], edge_index[0, EH:]
    dst_a, dst_b = edge_index[1, :EH], edge_index[1, EH:]
    src_ap, src_bp = _pad_idx(src_a), _pad_idx(src_b)
    dst_ap, dst_bp = _pad_idx(dst_a), _pad_idx(dst_b)
    he_a, he_b = h_edge[:EH], h_edge[EH:]
    zeros = jnp.zeros((N_NODES, D), jnp.float32)
    num_convs = We1.shape[0]

    a_proj, b_proj = _proj_call(h_node, We1[0, :D], We1[0, D:2 * D])
    for i in range(num_convs):
        ew = (We1[i, 2 * D:], be1[i].reshape(1, D), We2[i],
              be2[i].reshape(1, D), ge[i].reshape(1, D), bbe[i].reshape(1, D))
        g1a, g2a = _gather_half(a_proj, b_proj, src_ap, dst_ap)
        g1b, g2b = _gather_half(a_proj, b_proj, src_bp, dst_bp)
        he_a = _edge_call(g1a, g2a, he_a, *ew)
        he_b = _edge_call(g1b, g2b, he_b, *ew)
        partials = _scatter_call(he_a, he_b, dst_a, dst_b, zeros)
        j = min(i + 1, num_convs - 1)
        h_node, a_proj, b_proj = _node_call(
            h_node, partials[0], partials[1],
            Wn1[i, :D], Wn1[i, D:], bn1[i].reshape(1, D),
            Wn2[i], bn2[i].reshape(1, D),
            gn[i].reshape(1, D), bbn[i].reshape(1, D),
            We1[j, :D], We1[j, D:2 * D])
    return h_node, jnp.concatenate([he_a, he_b], axis=0)


# speedup vs baseline: 1.6235x; 1.3300x over previous
import functools

import jax
import jax.numpy as jnp
from jax import lax
from jax.experimental import pallas as pl
from jax.experimental.pallas import tpu as pltpu
from jax.experimental.pallas import tpu_sc as plsc

N_NODES = 10000
N_EDGES = 160000
D = 128
EH = N_EDGES // 2

NC = 2
NS = 16
NW = NC * NS
CHUNK = 128
ROWS_PER_TILE = 624
ROWS_TAIL = N_NODES - NS * ROWS_PER_TILE

_mesh = plsc.VectorSubcoreMesh(core_axis_name="c", subcore_axis_name="s")


def _make_gather(n_edges):
    nchunks = n_edges // CHUNK

    def body(a_hbm, b_hbm, src_hbm, dst_hbm, o1_hbm,
             sidx, didx, rows_a, sem_a):
        c = lax.axis_index("c")
        s = lax.axis_index("s")
        wid = s * NC + c
        n_my = (nchunks - wid + NW - 1) // NW

        def step(k, carry):
            base = (wid + k * NW) * CHUNK
            pltpu.sync_copy(src_hbm.at[pl.ds(base, CHUNK)], sidx)
            pltpu.sync_copy(dst_hbm.at[pl.ds(base, CHUNK)], didx)
            cp_a = pltpu.async_copy(a_hbm.at[sidx], rows_a, sem_a)
            cp_a.wait()
            pltpu.sync_copy(b_hbm.at[didx], rows_a, add=True)
            pltpu.sync_copy(rows_a, o1_hbm.at[pl.ds(base, CHUNK)])
            return carry

        lax.fori_loop(0, n_my, step, 0)

    return functools.partial(
        pl.kernel,
        out_type=jax.ShapeDtypeStruct((n_edges, D), jnp.float32),
        mesh=_mesh,
        scratch_types=[
            pltpu.VMEM((CHUNK,), jnp.int32),
            pltpu.VMEM((CHUNK,), jnp.int32),
            pltpu.VMEM((CHUNK, D), jnp.float32),
            pltpu.SemaphoreType.DMA,
        ],
    )(body)


_gather_half = _make_gather(EH)


def _scatter_body(ea_hbm, eb_hbm, dsta_hbm, dstb_hbm, zero_hbm, out_hbm,
                  didx, rows_e, shared):
    c = lax.axis_index("c")
    s = lax.axis_index("s")
    wid = s * NC + c
    row0 = s * ROWS_PER_TILE
    tail0 = NS * ROWS_PER_TILE
    pltpu.sync_copy(zero_hbm.at[pl.ds(row0, ROWS_PER_TILE)],
                    shared.at[pl.ds(row0, ROWS_PER_TILE)])

    @pl.when(s == NS - 1)
    def _():
        pltpu.sync_copy(zero_hbm.at[pl.ds(tail0, ROWS_TAIL)],
                        shared.at[pl.ds(tail0, ROWS_TAIL)])

    plsc.subcore_barrier()

    nchunks = EH // CHUNK
    n_my = (nchunks - wid + NW - 1) // NW
    for e_hbm, dst_hbm in ((ea_hbm, dsta_hbm), (eb_hbm, dstb_hbm)):
        def step(k, carry, e_hbm=e_hbm, dst_hbm=dst_hbm):
            base = (wid + k * NW) * CHUNK
            pltpu.sync_copy(dst_hbm.at[pl.ds(base, CHUNK)], didx)
            pltpu.sync_copy(e_hbm.at[pl.ds(base, CHUNK)], rows_e)
            pltpu.sync_copy(rows_e, shared.at[didx], add=True)
            return carry

        lax.fori_loop(0, n_my, step, 0)

    plsc.subcore_barrier()
    pltpu.sync_copy(shared.at[pl.ds(row0, ROWS_PER_TILE)],
                    out_hbm.at[c, pl.ds(row0, ROWS_PER_TILE)])

    @pl.when(s == NS - 1)
    def _():
        pltpu.sync_copy(shared.at[pl.ds(tail0, ROWS_TAIL)],
                        out_hbm.at[c, pl.ds(tail0, ROWS_TAIL)])


_scatter_call = functools.partial(
    pl.kernel,
    out_type=jax.ShapeDtypeStruct((NC, N_NODES, D), jnp.float32),
    mesh=_mesh,
    scratch_types=[
        pltpu.VMEM((CHUNK,), jnp.int32),
        pltpu.VMEM((CHUNK, D), jnp.float32),
        pltpu.VMEM_SHARED((N_NODES, D), jnp.float32),
    ],
)(_scatter_body)


BE = 2000
BN = 2000


def _proj_body(hn, ws, wd, out_a, out_b):
    x = hn[...]
    out_a[...] = jnp.dot(x, ws[...], preferred_element_type=jnp.float32)
    out_b[...] = jnp.dot(x, wd[...], preferred_element_type=jnp.float32)


def _edge_body(g12, he, w1, b1, w2, b2, g, bb, out):
    he_v = he[...]
    x = g12[...] + b1[...] + jnp.dot(
        he_v, w1[...], preferred_element_type=jnp.float32)
    h = jnp.maximum(x, 0.0)
    y = jnp.dot(h, w2[...], preferred_element_type=jnp.float32) + b2[...]
    mu = jnp.mean(y, axis=-1, keepdims=True)
    yc = y - mu
    var = jnp.mean(yc * yc, axis=-1, keepdims=True)
    out[...] = he_v + yc * lax.rsqrt(var + 1e-5) * g[...] + bb[...]


def _node_body(hn, p0, p1, w1a, w1b, b1, w2, b2, g, bb, ws, wd,
               out_h, out_a, out_b):
    hn_v = hn[...]
    agg = p0[...] + p1[...]
    x = (jnp.dot(hn_v, w1a[...], preferred_element_type=jnp.float32)
         + jnp.dot(agg, w1b[...], preferred_element_type=jnp.float32)
         + b1[...])
    h = jnp.maximum(x, 0.0)
    y = jnp.dot(h, w2[...], preferred_element_type=jnp.float32) + b2[...]
    mu = jnp.mean(y, axis=-1, keepdims=True)
    yc = y - mu
    var = jnp.mean(yc * yc, axis=-1, keepdims=True)
    hn_new = hn_v + yc * lax.rsqrt(var + 1e-5) * g[...] + bb[...]
    out_h[...] = hn_new
    out_a[...] = jnp.dot(hn_new, ws[...], preferred_element_type=jnp.float32)
    out_b[...] = jnp.dot(hn_new, wd[...], preferred_element_type=jnp.float32)


def _row_spec(bs):
    return pl.BlockSpec((bs, D), lambda i: (i, 0))


def _w_spec():
    return pl.BlockSpec((D, D), lambda i: (0, 0))


def _b_spec():
    return pl.BlockSpec((1, D), lambda i: (0, 0))


_proj_call = pl.pallas_call(
    _proj_body,
    grid=(N_NODES // BN,),
    in_specs=[_row_spec(BN), _w_spec(), _w_spec()],
    out_specs=[_row_spec(BN), _row_spec(BN)],
    out_shape=[jax.ShapeDtypeStruct((N_NODES, D), jnp.float32)] * 2,
)

_edge_call = pl.pallas_call(
    _edge_body,
    grid=(EH // BE,),
    in_specs=[_row_spec(BE), _row_spec(BE),
              _w_spec(), _b_spec(), _w_spec(), _b_spec(),
              _b_spec(), _b_spec()],
    out_specs=_row_spec(BE),
    out_shape=jax.ShapeDtypeStruct((EH, D), jnp.float32),
)

_node_call = pl.pallas_call(
    _node_body,
    grid=(N_NODES // BN,),
    in_specs=[_row_spec(BN), _row_spec(BN), _row_spec(BN),
              _w_spec(), _w_spec(), _b_spec(), _w_spec(), _b_spec(),
              _b_spec(), _b_spec(), _w_spec(), _w_spec()],
    out_specs=[_row_spec(BN), _row_spec(BN), _row_spec(BN)],
    out_shape=[jax.ShapeDtypeStruct((N_NODES, D), jnp.float32)] * 3,
)


def kernel(h_node, h_edge, edge_index, We1, be1, We2, be2, ge, bbe,
           Wn1, bn1, Wn2, bn2, gn, bbn):
    src_a, src_b = edge_index[0, :EH], edge_index[0, EH:]
    dst_a, dst_b = edge_index[1, :EH], edge_index[1, EH:]
    he_a, he_b = h_edge[:EH], h_edge[EH:]
    zeros = jnp.zeros((N_NODES, D), jnp.float32)
    num_convs = We1.shape[0]

    a_proj, b_proj = _proj_call(h_node, We1[0, :D], We1[0, D:2 * D])
    for i in range(num_convs):
        ew = (We1[i, 2 * D:], be1[i].reshape(1, D), We2[i],
              be2[i].reshape(1, D), ge[i].reshape(1, D), bbe[i].reshape(1, D))
        g_a = _gather_half(a_proj, b_proj, src_a, dst_a)
        g_b = _gather_half(a_proj, b_proj, src_b, dst_b)
        he_a = _edge_call(g_a, he_a, *ew)
        he_b = _edge_call(g_b, he_b, *ew)
        partials = _scatter_call(he_a, he_b, dst_a, dst_b, zeros)
        j = min(i + 1, num_convs - 1)
        h_node, a_proj, b_proj = _node_call(
            h_node, partials[0], partials[1],
            Wn1[i, :D], Wn1[i, D:], bn1[i].reshape(1, D),
            Wn2[i], bn2[i].reshape(1, D),
            gn[i].reshape(1, D), bbn[i].reshape(1, D),
            We1[j, :D], We1[j, D:2 * D])
    return h_node, jnp.concatenate([he_a, he_b], axis=0)


# speedup vs baseline: 1.7657x; 1.0876x over previous
import functools

import jax
import jax.numpy as jnp
from jax import lax
from jax.experimental import pallas as pl
from jax.experimental.pallas import tpu as pltpu
from jax.experimental.pallas import tpu_sc as plsc

N_NODES = 10000
N_EDGES = 160000
D = 128
EH = N_EDGES // 2

NC = 2
NS = 16
NW = NC * NS
CHUNK = 128
ROWS_PER_TILE = 624
ROWS_TAIL = N_NODES - NS * ROWS_PER_TILE

_mesh = plsc.VectorSubcoreMesh(core_axis_name="c", subcore_axis_name="s")


NCH = EH // CHUNK
CPW = (NCH + NW - 1) // NW


def _make_gather(n_edges):
    nchunks = n_edges // CHUNK

    def body(a_hbm, b_hbm, idxp_hbm, o1_hbm, idx_all, rows_a, sem_a):
        c = lax.axis_index("c")
        s = lax.axis_index("s")
        wid = s * NC + c
        n_my = (nchunks - wid + NW - 1) // NW
        pltpu.sync_copy(idxp_hbm.at[wid], idx_all)

        def step(k, carry):
            base = (wid + k * NW) * CHUNK
            cp_a = pltpu.async_copy(
                a_hbm.at[idx_all.at[pl.ds(2 * k * CHUNK, CHUNK)]],
                rows_a, sem_a)
            cp_a.wait()
            pltpu.sync_copy(
                b_hbm.at[idx_all.at[pl.ds((2 * k + 1) * CHUNK, CHUNK)]],
                rows_a, add=True)
            pltpu.sync_copy(rows_a, o1_hbm.at[pl.ds(base, CHUNK)])
            return carry

        lax.fori_loop(0, n_my, step, 0)

    return functools.partial(
        pl.kernel,
        out_type=jax.ShapeDtypeStruct((n_edges, D), jnp.float32),
        mesh=_mesh,
        scratch_types=[
            pltpu.VMEM((2 * CPW * CHUNK,), jnp.int32),
            pltpu.VMEM((CHUNK, D), jnp.float32),
            pltpu.SemaphoreType.DMA,
        ],
    )(body)


def _permute_idx(src, dst):
    pad = NW * CPW * CHUNK - EH
    s = jnp.concatenate([src, jnp.zeros((pad,), jnp.int32)])
    d = jnp.concatenate([dst, jnp.zeros((pad,), jnp.int32)])
    s = s.reshape(CPW, NW, CHUNK).transpose(1, 0, 2)
    d = d.reshape(CPW, NW, CHUNK).transpose(1, 0, 2)
    inter = jnp.stack([s, d], axis=2)
    return inter.reshape(NW, 2 * CPW * CHUNK)


_gather_half = _make_gather(EH)


def _scatter_body(ea_hbm, eb_hbm, dsta_hbm, dstb_hbm, zero_hbm, out_hbm,
                  didx, rows_e, shared):
    c = lax.axis_index("c")
    s = lax.axis_index("s")
    wid = s * NC + c
    row0 = s * ROWS_PER_TILE
    tail0 = NS * ROWS_PER_TILE
    pltpu.sync_copy(zero_hbm.at[pl.ds(row0, ROWS_PER_TILE)],
                    shared.at[pl.ds(row0, ROWS_PER_TILE)])

    @pl.when(s == NS - 1)
    def _():
        pltpu.sync_copy(zero_hbm.at[pl.ds(tail0, ROWS_TAIL)],
                        shared.at[pl.ds(tail0, ROWS_TAIL)])

    plsc.subcore_barrier()

    nchunks = EH // CHUNK
    n_my = (nchunks - wid + NW - 1) // NW
    for e_hbm, dst_hbm in ((ea_hbm, dsta_hbm), (eb_hbm, dstb_hbm)):
        def step(k, carry, e_hbm=e_hbm, dst_hbm=dst_hbm):
            base = (wid + k * NW) * CHUNK
            pltpu.sync_copy(dst_hbm.at[pl.ds(base, CHUNK)], didx)
            pltpu.sync_copy(e_hbm.at[pl.ds(base, CHUNK)], rows_e)
            pltpu.sync_copy(rows_e, shared.at[didx], add=True)
            return carry

        lax.fori_loop(0, n_my, step, 0)

    plsc.subcore_barrier()
    pltpu.sync_copy(shared.at[pl.ds(row0, ROWS_PER_TILE)],
                    out_hbm.at[c, pl.ds(row0, ROWS_PER_TILE)])

    @pl.when(s == NS - 1)
    def _():
        pltpu.sync_copy(shared.at[pl.ds(tail0, ROWS_TAIL)],
                        out_hbm.at[c, pl.ds(tail0, ROWS_TAIL)])


_scatter_call = functools.partial(
    pl.kernel,
    out_type=jax.ShapeDtypeStruct((NC, N_NODES, D), jnp.float32),
    mesh=_mesh,
    scratch_types=[
        pltpu.VMEM((CHUNK,), jnp.int32),
        pltpu.VMEM((CHUNK, D), jnp.float32),
        pltpu.VMEM_SHARED((N_NODES, D), jnp.float32),
    ],
)(_scatter_body)


BE = 2000
BN = 2000


def _proj_body(hn, ws, wd, out_a, out_b):
    x = hn[...]
    out_a[...] = jnp.dot(x, ws[...], preferred_element_type=jnp.float32)
    out_b[...] = jnp.dot(x, wd[...], preferred_element_type=jnp.float32)


def _edge_body(g12, he, w1, b1, w2, b2, g, bb, out):
    he_v = he[...]
    x = g12[...] + b1[...] + jnp.dot(
        he_v, w1[...], preferred_element_type=jnp.float32)
    h = jnp.maximum(x, 0.0)
    y = jnp.dot(h, w2[...], preferred_element_type=jnp.float32) + b2[...]
    mu = jnp.mean(y, axis=-1, keepdims=True)
    yc = y - mu
    var = jnp.mean(yc * yc, axis=-1, keepdims=True)
    out[...] = he_v + yc * lax.rsqrt(var + 1e-5) * g[...] + bb[...]


def _node_body(hn, p0, p1, w1a, w1b, b1, w2, b2, g, bb, ws, wd,
               out_h, out_a, out_b):
    hn_v = hn[...]
    agg = p0[...] + p1[...]
    x = (jnp.dot(hn_v, w1a[...], preferred_element_type=jnp.float32)
         + jnp.dot(agg, w1b[...], preferred_element_type=jnp.float32)
         + b1[...])
    h = jnp.maximum(x, 0.0)
    y = jnp.dot(h, w2[...], preferred_element_type=jnp.float32) + b2[...]
    mu = jnp.mean(y, axis=-1, keepdims=True)
    yc = y - mu
    var = jnp.mean(yc * yc, axis=-1, keepdims=True)
    hn_new = hn_v + yc * lax.rsqrt(var + 1e-5) * g[...] + bb[...]
    out_h[...] = hn_new
    out_a[...] = jnp.dot(hn_new, ws[...], preferred_element_type=jnp.float32)
    out_b[...] = jnp.dot(hn_new, wd[...], preferred_element_type=jnp.float32)


def _row_spec(bs):
    return pl.BlockSpec((bs, D), lambda i: (i, 0))


def _w_spec():
    return pl.BlockSpec((D, D), lambda i: (0, 0))


def _b_spec():
    return pl.BlockSpec((1, D), lambda i: (0, 0))


_proj_call = pl.pallas_call(
    _proj_body,
    grid=(N_NODES // BN,),
    in_specs=[_row_spec(BN), _w_spec(), _w_spec()],
    out_specs=[_row_spec(BN), _row_spec(BN)],
    out_shape=[jax.ShapeDtypeStruct((N_NODES, D), jnp.float32)] * 2,
)

_edge_call = pl.pallas_call(
    _edge_body,
    grid=(EH // BE,),
    in_specs=[_row_spec(BE), _row_spec(BE),
              _w_spec(), _b_spec(), _w_spec(), _b_spec(),
              _b_spec(), _b_spec()],
    out_specs=_row_spec(BE),
    out_shape=jax.ShapeDtypeStruct((EH, D), jnp.float32),
)

_node_call = pl.pallas_call(
    _node_body,
    grid=(N_NODES // BN,),
    in_specs=[_row_spec(BN), _row_spec(BN), _row_spec(BN),
              _w_spec(), _w_spec(), _b_spec(), _w_spec(), _b_spec(),
              _b_spec(), _b_spec(), _w_spec(), _w_spec()],
    out_specs=[_row_spec(BN), _row_spec(BN), _row_spec(BN)],
    out_shape=[jax.ShapeDtypeStruct((N_NODES, D), jnp.float32)] * 3,
)


def kernel(h_node, h_edge, edge_index, We1, be1, We2, be2, ge, bbe,
           Wn1, bn1, Wn2, bn2, gn, bbn):
    src_a, src_b = edge_index[0, :EH], edge_index[0, EH:]
    dst_a, dst_b = edge_index[1, :EH], edge_index[1, EH:]
    idxp_a = _permute_idx(src_a, dst_a)
    idxp_b = _permute_idx(src_b, dst_b)
    he_a, he_b = h_edge[:EH], h_edge[EH:]
    zeros = jnp.zeros((N_NODES, D), jnp.float32)
    num_convs = We1.shape[0]

    a_proj, b_proj = _proj_call(h_node, We1[0, :D], We1[0, D:2 * D])
    for i in range(num_convs):
        ew = (We1[i, 2 * D:], be1[i].reshape(1, D), We2[i],
              be2[i].reshape(1, D), ge[i].reshape(1, D), bbe[i].reshape(1, D))
        g_a = _gather_half(a_proj, b_proj, idxp_a)
        g_b = _gather_half(a_proj, b_proj, idxp_b)
        he_a = _edge_call(g_a, he_a, *ew)
        he_b = _edge_call(g_b, he_b, *ew)
        partials = _scatter_call(he_a, he_b, dst_a, dst_b, zeros)
        j = min(i + 1, num_convs - 1)
        h_node, a_proj, b_proj = _node_call(
            h_node, partials[0], partials[1],
            Wn1[i, :D], Wn1[i, D:], bn1[i].reshape(1, D),
            Wn2[i], bn2[i].reshape(1, D),
            gn[i].reshape(1, D), bbn[i].reshape(1, D),
            We1[j, :D], We1[j, D:2 * D])
    return h_node, jnp.concatenate([he_a, he_b], axis=0)


# speedup vs baseline: 1.8483x; 1.0468x over previous
import functools

import jax
import jax.numpy as jnp
from jax import lax
from jax.experimental import pallas as pl
from jax.experimental.pallas import tpu as pltpu
from jax.experimental.pallas import tpu_sc as plsc

N_NODES = 10000
N_EDGES = 160000
D = 128
EH = N_EDGES // 2

NC = 2
NS = 16
NW = NC * NS
CHUNK = 128
ROWS_PER_TILE = 624
ROWS_TAIL = N_NODES - NS * ROWS_PER_TILE

_mesh = plsc.VectorSubcoreMesh(core_axis_name="c", subcore_axis_name="s")


NCH = EH // CHUNK
CPW = (NCH + NW - 1) // NW


def _make_gather(n_edges):
    nchunks = n_edges // CHUNK

    def body(a_hbm, b_hbm, idxp_hbm, o1_hbm, idx_all, rows_a, sem_a):
        c = lax.axis_index("c")
        s = lax.axis_index("s")
        wid = s * NC + c
        n_my = (nchunks - wid + NW - 1) // NW
        pltpu.sync_copy(idxp_hbm.at[wid], idx_all)

        def step(k, carry):
            base = (wid + k * NW) * CHUNK
            cp_a = pltpu.async_copy(
                a_hbm.at[idx_all.at[pl.ds(2 * k * CHUNK, CHUNK)]],
                rows_a, sem_a)
            cp_a.wait()
            pltpu.sync_copy(
                b_hbm.at[idx_all.at[pl.ds((2 * k + 1) * CHUNK, CHUNK)]],
                rows_a, add=True)
            pltpu.sync_copy(rows_a, o1_hbm.at[pl.ds(base, CHUNK)])
            return carry

        lax.fori_loop(0, n_my, step, 0)

    return functools.partial(
        pl.kernel,
        out_type=jax.ShapeDtypeStruct((n_edges, D), jnp.float32),
        mesh=_mesh,
        scratch_types=[
            pltpu.VMEM((2 * CPW * CHUNK,), jnp.int32),
            pltpu.VMEM((CHUNK, D), jnp.float32),
            pltpu.SemaphoreType.DMA,
        ],
    )(body)


def _permute_idx(src, dst):
    pad = NW * CPW * CHUNK - EH
    s = jnp.concatenate([src, jnp.zeros((pad,), jnp.int32)])
    d = jnp.concatenate([dst, jnp.zeros((pad,), jnp.int32)])
    s = s.reshape(CPW, NW, CHUNK).transpose(1, 0, 2)
    d = d.reshape(CPW, NW, CHUNK).transpose(1, 0, 2)
    inter = jnp.stack([s, d], axis=2)
    return inter.reshape(NW, 2 * CPW * CHUNK)


_gather_half = _make_gather(EH)


def _scatter_body(ea_hbm, eb_hbm, idxpa_hbm, idxpb_hbm, zero_hbm, out_hbm,
                  idx_all, rows_e, shared):
    c = lax.axis_index("c")
    s = lax.axis_index("s")
    wid = s * NC + c
    row0 = s * ROWS_PER_TILE
    tail0 = NS * ROWS_PER_TILE
    pltpu.sync_copy(zero_hbm.at[pl.ds(row0, ROWS_PER_TILE)],
                    shared.at[pl.ds(row0, ROWS_PER_TILE)])

    @pl.when(s == NS - 1)
    def _():
        pltpu.sync_copy(zero_hbm.at[pl.ds(tail0, ROWS_TAIL)],
                        shared.at[pl.ds(tail0, ROWS_TAIL)])

    plsc.subcore_barrier()

    nchunks = EH // CHUNK
    n_my = (nchunks - wid + NW - 1) // NW
    for e_hbm, idxp_hbm in ((ea_hbm, idxpa_hbm), (eb_hbm, idxpb_hbm)):
        pltpu.sync_copy(idxp_hbm.at[wid], idx_all)

        def step(k, carry, e_hbm=e_hbm):
            base = (wid + k * NW) * CHUNK
            pltpu.sync_copy(e_hbm.at[pl.ds(base, CHUNK)], rows_e)
            pltpu.sync_copy(
                rows_e,
                shared.at[idx_all.at[pl.ds((2 * k + 1) * CHUNK, CHUNK)]],
                add=True)
            return carry

        lax.fori_loop(0, n_my, step, 0)

    plsc.subcore_barrier()
    pltpu.sync_copy(shared.at[pl.ds(row0, ROWS_PER_TILE)],
                    out_hbm.at[c, pl.ds(row0, ROWS_PER_TILE)])

    @pl.when(s == NS - 1)
    def _():
        pltpu.sync_copy(shared.at[pl.ds(tail0, ROWS_TAIL)],
                        out_hbm.at[c, pl.ds(tail0, ROWS_TAIL)])


_scatter_call = functools.partial(
    pl.kernel,
    out_type=jax.ShapeDtypeStruct((NC, N_NODES, D), jnp.float32),
    mesh=_mesh,
    scratch_types=[
        pltpu.VMEM((2 * CPW * CHUNK,), jnp.int32),
        pltpu.VMEM((CHUNK, D), jnp.float32),
        pltpu.VMEM_SHARED((N_NODES, D), jnp.float32),
    ],
)(_scatter_body)


BE = 2000
BN = 2000


def _proj_body(hn, ws, wd, out_a, out_b):
    x = hn[...]
    out_a[...] = jnp.dot(x, ws[...], preferred_element_type=jnp.float32)
    out_b[...] = jnp.dot(x, wd[...], preferred_element_type=jnp.float32)


def _edge_body(g12, he, w1, b1, w2, b2, g, bb, out):
    he_v = he[...]
    x = g12[...] + b1[...] + jnp.dot(
        he_v, w1[...], preferred_element_type=jnp.float32)
    h = jnp.maximum(x, 0.0)
    y = jnp.dot(h, w2[...], preferred_element_type=jnp.float32) + b2[...]
    mu = jnp.mean(y, axis=-1, keepdims=True)
    yc = y - mu
    var = jnp.mean(yc * yc, axis=-1, keepdims=True)
    out[...] = he_v + yc * lax.rsqrt(var + 1e-5) * g[...] + bb[...]


def _node_body(hn, p0, p1, w1a, w1b, b1, w2, b2, g, bb, ws, wd,
               out_h, out_a, out_b):
    hn_v = hn[...]
    agg = p0[...] + p1[...]
    x = (jnp.dot(hn_v, w1a[...], preferred_element_type=jnp.float32)
         + jnp.dot(agg, w1b[...], preferred_element_type=jnp.float32)
         + b1[...])
    h = jnp.maximum(x, 0.0)
    y = jnp.dot(h, w2[...], preferred_element_type=jnp.float32) + b2[...]
    mu = jnp.mean(y, axis=-1, keepdims=True)
    yc = y - mu
    var = jnp.mean(yc * yc, axis=-1, keepdims=True)
    hn_new = hn_v + yc * lax.rsqrt(var + 1e-5) * g[...] + bb[...]
    out_h[...] = hn_new
    out_a[...] = jnp.dot(hn_new, ws[...], preferred_element_type=jnp.float32)
    out_b[...] = jnp.dot(hn_new, wd[...], preferred_element_type=jnp.float32)


def _row_spec(bs):
    return pl.BlockSpec((bs, D), lambda i: (i, 0))


def _w_spec():
    return pl.BlockSpec((D, D), lambda i: (0, 0))


def _b_spec():
    return pl.BlockSpec((1, D), lambda i: (0, 0))


_proj_call = pl.pallas_call(
    _proj_body,
    grid=(N_NODES // BN,),
    in_specs=[_row_spec(BN), _w_spec(), _w_spec()],
    out_specs=[_row_spec(BN), _row_spec(BN)],
    out_shape=[jax.ShapeDtypeStruct((N_NODES, D), jnp.float32)] * 2,
)

_edge_call = pl.pallas_call(
    _edge_body,
    grid=(EH // BE,),
    in_specs=[_row_spec(BE), _row_spec(BE),
              _w_spec(), _b_spec(), _w_spec(), _b_spec(),
              _b_spec(), _b_spec()],
    out_specs=_row_spec(BE),
    out_shape=jax.ShapeDtypeStruct((EH, D), jnp.float32),
)

_node_call = pl.pallas_call(
    _node_body,
    grid=(N_NODES // BN,),
    in_specs=[_row_spec(BN), _row_spec(BN), _row_spec(BN),
              _w_spec(), _w_spec(), _b_spec(), _w_spec(), _b_spec(),
              _b_spec(), _b_spec(), _w_spec(), _w_spec()],
    out_specs=[_row_spec(BN), _row_spec(BN), _row_spec(BN)],
    out_shape=[jax.ShapeDtypeStruct((N_NODES, D), jnp.float32)] * 3,
)


def kernel(h_node, h_edge, edge_index, We1, be1, We2, be2, ge, bbe,
           Wn1, bn1, Wn2, bn2, gn, bbn):
    src_a, src_b = edge_index[0, :EH], edge_index[0, EH:]
    dst_a, dst_b = edge_index[1, :EH], edge_index[1, EH:]
    idxp_a = _permute_idx(src_a, dst_a)
    idxp_b = _permute_idx(src_b, dst_b)
    he_a, he_b = h_edge[:EH], h_edge[EH:]
    zeros = jnp.zeros((N_NODES, D), jnp.float32)
    num_convs = We1.shape[0]

    a_proj, b_proj = _proj_call(h_node, We1[0, :D], We1[0, D:2 * D])
    for i in range(num_convs):
        ew = (We1[i, 2 * D:], be1[i].reshape(1, D), We2[i],
              be2[i].reshape(1, D), ge[i].reshape(1, D), bbe[i].reshape(1, D))
        g_a = _gather_half(a_proj, b_proj, idxp_a)
        g_b = _gather_half(a_proj, b_proj, idxp_b)
        he_a = _edge_call(g_a, he_a, *ew)
        he_b = _edge_call(g_b, he_b, *ew)
        partials = _scatter_call(he_a, he_b, idxp_a, idxp_b, zeros)
        j = min(i + 1, num_convs - 1)
        h_node, a_proj, b_proj = _node_call(
            h_node, partials[0], partials[1],
            Wn1[i, :D], Wn1[i, D:], bn1[i].reshape(1, D),
            Wn2[i], bn2[i].reshape(1, D),
            gn[i].reshape(1, D), bbn[i].reshape(1, D),
            We1[j, :D], We1[j, D:2 * D])
    return h_node, jnp.concatenate([he_a, he_b], axis=0)


# speedup vs baseline: 1.9861x; 1.0745x over previous
import functools

import jax
import jax.numpy as jnp
from jax import lax
from jax.experimental import pallas as pl
from jax.experimental.pallas import tpu as pltpu
from jax.experimental.pallas import tpu_sc as plsc

N_NODES = 10000
N_EDGES = 160000
D = 128
EH = N_EDGES // 2

NC = 2
NS = 16
NW = NC * NS
CHUNK = 128
ROWS_PER_TILE = 624
ROWS_TAIL = N_NODES - NS * ROWS_PER_TILE

_mesh = plsc.VectorSubcoreMesh(core_axis_name="c", subcore_axis_name="s")


NCH = EH // CHUNK
CPW = (NCH + NW - 1) // NW


def _make_gather(n_edges):
    nchunks = n_edges // CHUNK

    def body(a_hbm, b_hbm, idxp_hbm, o1_hbm, idx_all, rows0, rows1,
             sem0, sem1):
        c = lax.axis_index("c")
        s = lax.axis_index("s")
        wid = s * NC + c
        n_my = (nchunks - wid + NW - 1) // NW
        rows = (rows0, rows1)
        sems = (sem0, sem1)
        pltpu.sync_copy(idxp_hbm.at[wid], idx_all)

        def start_a(k, b):
            pltpu.async_copy(
                a_hbm.at[idx_all.at[pl.ds(2 * k * CHUNK, CHUNK)]],
                rows[b], sems[b])

        start_a(0, 0)
        for k in range(CPW):
            b = k % 2

            @pl.when(k < n_my)
            def _(k=k, b=b):
                pltpu.make_async_copy(a_hbm.at[pl.ds(0, CHUNK)],
                                      rows[b], sems[b]).wait()
                if k + 1 < CPW:
                    @pl.when(k + 1 < n_my)
                    def _():
                        start_a(k + 1, 1 - b)
                pltpu.sync_copy(
                    b_hbm.at[idx_all.at[pl.ds((2 * k + 1) * CHUNK, CHUNK)]],
                    rows[b], add=True)
                pltpu.sync_copy(rows[b],
                                o1_hbm.at[pl.ds((wid + k * NW) * CHUNK,
                                                CHUNK)])

    return functools.partial(
        pl.kernel,
        out_type=jax.ShapeDtypeStruct((n_edges, D), jnp.float32),
        mesh=_mesh,
        scratch_types=[
            pltpu.VMEM((2 * CPW * CHUNK,), jnp.int32),
            pltpu.VMEM((CHUNK, D), jnp.float32),
            pltpu.VMEM((CHUNK, D), jnp.float32),
            pltpu.SemaphoreType.DMA,
            pltpu.SemaphoreType.DMA,
        ],
    )(body)


def _permute_idx(src, dst):
    pad = NW * CPW * CHUNK - EH
    s = jnp.concatenate([src, jnp.zeros((pad,), jnp.int32)])
    d = jnp.concatenate([dst, jnp.zeros((pad,), jnp.int32)])
    s = s.reshape(CPW, NW, CHUNK).transpose(1, 0, 2)
    d = d.reshape(CPW, NW, CHUNK).transpose(1, 0, 2)
    inter = jnp.stack([s, d], axis=2)
    return inter.reshape(NW, 2 * CPW * CHUNK)


_gather_half = _make_gather(EH)


def _scatter_body(ea_hbm, eb_hbm, idxpa_hbm, idxpb_hbm, zero_hbm, out_hbm,
                  idx_all, rows_e, shared):
    c = lax.axis_index("c")
    s = lax.axis_index("s")
    wid = s * NC + c
    row0 = s * ROWS_PER_TILE
    tail0 = NS * ROWS_PER_TILE
    pltpu.sync_copy(zero_hbm.at[pl.ds(row0, ROWS_PER_TILE)],
                    shared.at[pl.ds(row0, ROWS_PER_TILE)])

    @pl.when(s == NS - 1)
    def _():
        pltpu.sync_copy(zero_hbm.at[pl.ds(tail0, ROWS_TAIL)],
                        shared.at[pl.ds(tail0, ROWS_TAIL)])

    plsc.subcore_barrier()

    nchunks = EH // CHUNK
    n_my = (nchunks - wid + NW - 1) // NW
    for e_hbm, idxp_hbm in ((ea_hbm, idxpa_hbm), (eb_hbm, idxpb_hbm)):
        pltpu.sync_copy(idxp_hbm.at[wid], idx_all)

        def step(k, carry, e_hbm=e_hbm):
            base = (wid + k * NW) * CHUNK
            pltpu.sync_copy(e_hbm.at[pl.ds(base, CHUNK)], rows_e)
            pltpu.sync_copy(
                rows_e,
                shared.at[idx_all.at[pl.ds((2 * k + 1) * CHUNK, CHUNK)]],
                add=True)
            return carry

        lax.fori_loop(0, n_my, step, 0)

    plsc.subcore_barrier()
    pltpu.sync_copy(shared.at[pl.ds(row0, ROWS_PER_TILE)],
                    out_hbm.at[c, pl.ds(row0, ROWS_PER_TILE)])

    @pl.when(s == NS - 1)
    def _():
        pltpu.sync_copy(shared.at[pl.ds(tail0, ROWS_TAIL)],
                        out_hbm.at[c, pl.ds(tail0, ROWS_TAIL)])


_scatter_call = functools.partial(
    pl.kernel,
    out_type=jax.ShapeDtypeStruct((NC, N_NODES, D), jnp.float32),
    mesh=_mesh,
    scratch_types=[
        pltpu.VMEM((2 * CPW * CHUNK,), jnp.int32),
        pltpu.VMEM((CHUNK, D), jnp.float32),
        pltpu.VMEM_SHARED((N_NODES, D), jnp.float32),
    ],
)(_scatter_body)


BE = 2000
BN = 2000


def _proj_body(hn, ws, wd, out_a, out_b):
    x = hn[...]
    out_a[...] = jnp.dot(x, ws[...], preferred_element_type=jnp.float32)
    out_b[...] = jnp.dot(x, wd[...], preferred_element_type=jnp.float32)


def _edge_body(g12, he, w1, b1, w2, b2, g, bb, out):
    he_v = he[...]
    x = g12[...] + b1[...] + jnp.dot(
        he_v, w1[...], preferred_element_type=jnp.float32)
    h = jnp.maximum(x, 0.0)
    y = jnp.dot(h, w2[...], preferred_element_type=jnp.float32) + b2[...]
    mu = jnp.mean(y, axis=-1, keepdims=True)
    yc = y - mu
    var = jnp.mean(yc * yc, axis=-1, keepdims=True)
    out[...] = he_v + yc * lax.rsqrt(var + 1e-5) * g[...] + bb[...]


def _node_body(hn, p0, p1, w1a, w1b, b1, w2, b2, g, bb, ws, wd,
               out_h, out_a, out_b):
    hn_v = hn[...]
    agg = p0[...] + p1[...]
    x = (jnp.dot(hn_v, w1a[...], preferred_element_type=jnp.float32)
         + jnp.dot(agg, w1b[...], preferred_element_type=jnp.float32)
         + b1[...])
    h = jnp.maximum(x, 0.0)
    y = jnp.dot(h, w2[...], preferred_element_type=jnp.float32) + b2[...]
    mu = jnp.mean(y, axis=-1, keepdims=True)
    yc = y - mu
    var = jnp.mean(yc * yc, axis=-1, keepdims=True)
    hn_new = hn_v + yc * lax.rsqrt(var + 1e-5) * g[...] + bb[...]
    out_h[...] = hn_new
    out_a[...] = jnp.dot(hn_new, ws[...], preferred_element_type=jnp.float32)
    out_b[...] = jnp.dot(hn_new, wd[...], preferred_element_type=jnp.float32)


def _row_spec(bs):
    return pl.BlockSpec((bs, D), lambda i: (i, 0))


def _w_spec():
    return pl.BlockSpec((D, D), lambda i: (0, 0))


def _b_spec():
    return pl.BlockSpec((1, D), lambda i: (0, 0))


_proj_call = pl.pallas_call(
    _proj_body,
    grid=(N_NODES // BN,),
    in_specs=[_row_spec(BN), _w_spec(), _w_spec()],
    out_specs=[_row_spec(BN), _row_spec(BN)],
    out_shape=[jax.ShapeDtypeStruct((N_NODES, D), jnp.float32)] * 2,
)

_edge_call = pl.pallas_call(
    _edge_body,
    grid=(EH // BE,),
    in_specs=[_row_spec(BE), _row_spec(BE),
              _w_spec(), _b_spec(), _w_spec(), _b_spec(),
              _b_spec(), _b_spec()],
    out_specs=_row_spec(BE),
    out_shape=jax.ShapeDtypeStruct((EH, D), jnp.float32),
)

_node_call = pl.pallas_call(
    _node_body,
    grid=(N_NODES // BN,),
    in_specs=[_row_spec(BN), _row_spec(BN), _row_spec(BN),
              _w_spec(), _w_spec(), _b_spec(), _w_spec(), _b_spec(),
              _b_spec(), _b_spec(), _w_spec(), _w_spec()],
    out_specs=[_row_spec(BN), _row_spec(BN), _row_spec(BN)],
    out_shape=[jax.ShapeDtypeStruct((N_NODES, D), jnp.float32)] * 3,
)


def kernel(h_node, h_edge, edge_index, We1, be1, We2, be2, ge, bbe,
           Wn1, bn1, Wn2, bn2, gn, bbn):
    src_a, src_b = edge_index[0, :EH], edge_index[0, EH:]
    dst_a, dst_b = edge_index[1, :EH], edge_index[1, EH:]
    idxp_a = _permute_idx(src_a, dst_a)
    idxp_b = _permute_idx(src_b, dst_b)
    he_a, he_b = h_edge[:EH], h_edge[EH:]
    zeros = jnp.zeros((N_NODES, D), jnp.float32)
    num_convs = We1.shape[0]

    a_proj, b_proj = _proj_call(h_node, We1[0, :D], We1[0, D:2 * D])
    for i in range(num_convs):
        ew = (We1[i, 2 * D:], be1[i].reshape(1, D), We2[i],
              be2[i].reshape(1, D), ge[i].reshape(1, D), bbe[i].reshape(1, D))
        g_a = _gather_half(a_proj, b_proj, idxp_a)
        g_b = _gather_half(a_proj, b_proj, idxp_b)
        he_a = _edge_call(g_a, he_a, *ew)
        he_b = _edge_call(g_b, he_b, *ew)
        partials = _scatter_call(he_a, he_b, idxp_a, idxp_b, zeros)
        j = min(i + 1, num_convs - 1)
        h_node, a_proj, b_proj = _node_call(
            h_node, partials[0], partials[1],
            Wn1[i, :D], Wn1[i, D:], bn1[i].reshape(1, D),
            Wn2[i], bn2[i].reshape(1, D),
            gn[i].reshape(1, D), bbn[i].reshape(1, D),
            We1[j, :D], We1[j, D:2 * D])
    return h_node, jnp.concatenate([he_a, he_b], axis=0)


# speedup vs baseline: 2.1184x; 1.0666x over previous
import functools

import jax
import jax.numpy as jnp
from jax import lax
from jax.experimental import pallas as pl
from jax.experimental.pallas import tpu as pltpu
from jax.experimental.pallas import tpu_sc as plsc

N_NODES = 10000
N_EDGES = 160000
D = 128
EH = N_EDGES // 2

NC = 2
NS = 16
NW = NC * NS
CHUNK = 128
ROWS_PER_TILE = 624
ROWS_TAIL = N_NODES - NS * ROWS_PER_TILE

_mesh = plsc.VectorSubcoreMesh(core_axis_name="c", subcore_axis_name="s")


NCH = EH // CHUNK
CPW = (NCH + NW - 1) // NW


def _make_gather(n_edges):
    nchunks = n_edges // CHUNK

    def body(a_hbm, b_hbm, idxp_hbm, o1_hbm, idx_all, rows0, rows1,
             sem0, sem1):
        c = lax.axis_index("c")
        s = lax.axis_index("s")
        wid = s * NC + c
        n_my = (nchunks - wid + NW - 1) // NW
        rows = (rows0, rows1)
        sems = (sem0, sem1)
        pltpu.sync_copy(idxp_hbm.at[wid], idx_all)

        def start_a(k, b):
            pltpu.async_copy(
                a_hbm.at[idx_all.at[pl.ds(2 * k * CHUNK, CHUNK)]],
                rows[b], sems[b])

        start_a(0, 0)
        for k in range(CPW):
            b = k % 2

            @pl.when(k < n_my)
            def _(k=k, b=b):
                pltpu.make_async_copy(a_hbm.at[pl.ds(0, CHUNK)],
                                      rows[b], sems[b]).wait()
                if k + 1 < CPW:
                    @pl.when(k + 1 < n_my)
                    def _():
                        start_a(k + 1, 1 - b)
                pltpu.sync_copy(
                    b_hbm.at[idx_all.at[pl.ds((2 * k + 1) * CHUNK, CHUNK)]],
                    rows[b], add=True)
                pltpu.sync_copy(rows[b],
                                o1_hbm.at[pl.ds((wid + k * NW) * CHUNK,
                                                CHUNK)])

    return functools.partial(
        pl.kernel,
        out_type=jax.ShapeDtypeStruct((n_edges, D), jnp.float32),
        mesh=_mesh,
        scratch_types=[
            pltpu.VMEM((2 * CPW * CHUNK,), jnp.int32),
            pltpu.VMEM((CHUNK, D), jnp.float32),
            pltpu.VMEM((CHUNK, D), jnp.float32),
            pltpu.SemaphoreType.DMA,
            pltpu.SemaphoreType.DMA,
        ],
    )(body)


def _permute_idx(src, dst):
    pad = NW * CPW * CHUNK - EH
    s = jnp.concatenate([src, jnp.zeros((pad,), jnp.int32)])
    d = jnp.concatenate([dst, jnp.zeros((pad,), jnp.int32)])
    s = s.reshape(CPW, NW, CHUNK).transpose(1, 0, 2)
    d = d.reshape(CPW, NW, CHUNK).transpose(1, 0, 2)
    inter = jnp.stack([s, d], axis=2)
    return inter.reshape(NW, 2 * CPW * CHUNK)


_gather_half = _make_gather(EH)


def _scatter_body(ea_hbm, eb_hbm, idxpa_hbm, idxpb_hbm, zero_hbm, out_hbm,
                  idx_all, rows0, rows1, sem0, sem1, shared):
    c = lax.axis_index("c")
    s = lax.axis_index("s")
    wid = s * NC + c
    row0 = s * ROWS_PER_TILE
    tail0 = NS * ROWS_PER_TILE
    rows = (rows0, rows1)
    sems = (sem0, sem1)
    pltpu.sync_copy(zero_hbm.at[pl.ds(row0, ROWS_PER_TILE)],
                    shared.at[pl.ds(row0, ROWS_PER_TILE)])

    @pl.when(s == NS - 1)
    def _():
        pltpu.sync_copy(zero_hbm.at[pl.ds(tail0, ROWS_TAIL)],
                        shared.at[pl.ds(tail0, ROWS_TAIL)])

    plsc.subcore_barrier()

    nchunks = EH // CHUNK
    n_my = (nchunks - wid + NW - 1) // NW
    for e_hbm, idxp_hbm in ((ea_hbm, idxpa_hbm), (eb_hbm, idxpb_hbm)):
        pltpu.sync_copy(idxp_hbm.at[wid], idx_all)

        def start_load(k, b, e_hbm=e_hbm):
            pltpu.async_copy(e_hbm.at[pl.ds((wid + k * NW) * CHUNK, CHUNK)],
                             rows[b], sems[b])

        start_load(0, 0)
        for k in range(CPW):
            b = k % 2

            @pl.when(k < n_my)
            def _(k=k, b=b, e_hbm=e_hbm, start_load=start_load):
                pltpu.make_async_copy(e_hbm.at[pl.ds(0, CHUNK)],
                                      rows[b], sems[b]).wait()
                if k + 1 < CPW:
                    @pl.when(k + 1 < n_my)
                    def _():
                        start_load(k + 1, 1 - b)
                pltpu.sync_copy(
                    rows[b],
                    shared.at[idx_all.at[pl.ds((2 * k + 1) * CHUNK, CHUNK)]],
                    add=True)

    plsc.subcore_barrier()
    pltpu.sync_copy(shared.at[pl.ds(row0, ROWS_PER_TILE)],
                    out_hbm.at[c, pl.ds(row0, ROWS_PER_TILE)])

    @pl.when(s == NS - 1)
    def _():
        pltpu.sync_copy(shared.at[pl.ds(tail0, ROWS_TAIL)],
                        out_hbm.at[c, pl.ds(tail0, ROWS_TAIL)])


_scatter_call = functools.partial(
    pl.kernel,
    out_type=jax.ShapeDtypeStruct((NC, N_NODES, D), jnp.float32),
    mesh=_mesh,
    scratch_types=[
        pltpu.VMEM((2 * CPW * CHUNK,), jnp.int32),
        pltpu.VMEM((CHUNK, D), jnp.float32),
        pltpu.VMEM((CHUNK, D), jnp.float32),
        pltpu.SemaphoreType.DMA,
        pltpu.SemaphoreType.DMA,
        pltpu.VMEM_SHARED((N_NODES, D), jnp.float32),
    ],
)(_scatter_body)


BE = 2000
BN = 2000


def _proj_body(hn, ws, wd, out_a, out_b):
    x = hn[...]
    out_a[...] = jnp.dot(x, ws[...], preferred_element_type=jnp.float32)
    out_b[...] = jnp.dot(x, wd[...], preferred_element_type=jnp.float32)


def _edge_body(g12, he, w1, b1, w2, b2, g, bb, out):
    he_v = he[...]
    x = g12[...] + b1[...] + jnp.dot(
        he_v, w1[...], preferred_element_type=jnp.float32)
    h = jnp.maximum(x, 0.0)
    y = jnp.dot(h, w2[...], preferred_element_type=jnp.float32) + b2[...]
    mu = jnp.mean(y, axis=-1, keepdims=True)
    yc = y - mu
    var = jnp.mean(yc * yc, axis=-1, keepdims=True)
    out[...] = he_v + yc * lax.rsqrt(var + 1e-5) * g[...] + bb[...]


def _node_body(hn, p0, p1, w1a, w1b, b1, w2, b2, g, bb, ws, wd,
               out_h, out_a, out_b):
    hn_v = hn[...]
    agg = p0[...] + p1[...]
    x = (jnp.dot(hn_v, w1a[...], preferred_element_type=jnp.float32)
         + jnp.dot(agg, w1b[...], preferred_element_type=jnp.float32)
         + b1[...])
    h = jnp.maximum(x, 0.0)
    y = jnp.dot(h, w2[...], preferred_element_type=jnp.float32) + b2[...]
    mu = jnp.mean(y, axis=-1, keepdims=True)
    yc = y - mu
    var = jnp.mean(yc * yc, axis=-1, keepdims=True)
    hn_new = hn_v + yc * lax.rsqrt(var + 1e-5) * g[...] + bb[...]
    out_h[...] = hn_new
    out_a[...] = jnp.dot(hn_new, ws[...], preferred_element_type=jnp.float32)
    out_b[...] = jnp.dot(hn_new, wd[...], preferred_element_type=jnp.float32)


def _row_spec(bs):
    return pl.BlockSpec((bs, D), lambda i: (i, 0))


def _w_spec():
    return pl.BlockSpec((D, D), lambda i: (0, 0))


def _b_spec():
    return pl.BlockSpec((1, D), lambda i: (0, 0))


_proj_call = pl.pallas_call(
    _proj_body,
    grid=(N_NODES // BN,),
    in_specs=[_row_spec(BN), _w_spec(), _w_spec()],
    out_specs=[_row_spec(BN), _row_spec(BN)],
    out_shape=[jax.ShapeDtypeStruct((N_NODES, D), jnp.float32)] * 2,
)

_edge_call = pl.pallas_call(
    _edge_body,
    grid=(EH // BE,),
    in_specs=[_row_spec(BE), _row_spec(BE),
              _w_spec(), _b_spec(), _w_spec(), _b_spec(),
              _b_spec(), _b_spec()],
    out_specs=_row_spec(BE),
    out_shape=jax.ShapeDtypeStruct((EH, D), jnp.float32),
)

_node_call = pl.pallas_call(
    _node_body,
    grid=(N_NODES // BN,),
    in_specs=[_row_spec(BN), _row_spec(BN), _row_spec(BN),
              _w_spec(), _w_spec(), _b_spec(), _w_spec(), _b_spec(),
              _b_spec(), _b_spec(), _w_spec(), _w_spec()],
    out_specs=[_row_spec(BN), _row_spec(BN), _row_spec(BN)],
    out_shape=[jax.ShapeDtypeStruct((N_NODES, D), jnp.float32)] * 3,
)


def kernel(h_node, h_edge, edge_index, We1, be1, We2, be2, ge, bbe,
           Wn1, bn1, Wn2, bn2, gn, bbn):
    src_a, src_b = edge_index[0, :EH], edge_index[0, EH:]
    dst_a, dst_b = edge_index[1, :EH], edge_index[1, EH:]
    idxp_a = _permute_idx(src_a, dst_a)
    idxp_b = _permute_idx(src_b, dst_b)
    he_a, he_b = h_edge[:EH], h_edge[EH:]
    zeros = jnp.zeros((N_NODES, D), jnp.float32)
    num_convs = We1.shape[0]

    a_proj, b_proj = _proj_call(h_node, We1[0, :D], We1[0, D:2 * D])
    for i in range(num_convs):
        ew = (We1[i, 2 * D:], be1[i].reshape(1, D), We2[i],
              be2[i].reshape(1, D), ge[i].reshape(1, D), bbe[i].reshape(1, D))
        g_a = _gather_half(a_proj, b_proj, idxp_a)
        g_b = _gather_half(a_proj, b_proj, idxp_b)
        he_a = _edge_call(g_a, he_a, *ew)
        he_b = _edge_call(g_b, he_b, *ew)
        partials = _scatter_call(he_a, he_b, idxp_a, idxp_b, zeros)
        j = min(i + 1, num_convs - 1)
        h_node, a_proj, b_proj = _node_call(
            h_node, partials[0], partials[1],
            Wn1[i, :D], Wn1[i, D:], bn1[i].reshape(1, D),
            Wn2[i], bn2[i].reshape(1, D),
            gn[i].reshape(1, D), bbn[i].reshape(1, D),
            We1[j, :D], We1[j, D:2 * D])
    return h_node, jnp.concatenate([he_a, he_b], axis=0)


# speedup vs baseline: 2.1268x; 1.0040x over previous
import functools

import jax
import jax.numpy as jnp
from jax import lax
from jax.experimental import pallas as pl
from jax.experimental.pallas import tpu as pltpu
from jax.experimental.pallas import tpu_sc as plsc

N_NODES = 10000
N_EDGES = 160000
D = 128
EH = N_EDGES // 2

NC = 2
NS = 16
NW = NC * NS
CHUNK = 128
ROWS_PER_TILE = 624
ROWS_TAIL = N_NODES - NS * ROWS_PER_TILE

_mesh = plsc.VectorSubcoreMesh(core_axis_name="c", subcore_axis_name="s")


NCH = EH // CHUNK
CPW = (NCH + NW - 1) // NW


def _make_gather(n_edges):
    nchunks = n_edges // CHUNK

    def body(a_hbm, b_hbm, idxp_hbm, o1_hbm, idx_all, rows0, rows1,
             sem0, sem1, semw0, semw1):
        c = lax.axis_index("c")
        s = lax.axis_index("s")
        wid = s * NC + c
        n_my = (nchunks - wid + NW - 1) // NW
        rows = (rows0, rows1)
        sems = (sem0, sem1)
        semws = (semw0, semw1)
        pltpu.sync_copy(idxp_hbm.at[wid], idx_all)

        def start_a(k, b):
            pltpu.async_copy(
                a_hbm.at[idx_all.at[pl.ds(2 * k * CHUNK, CHUNK)]],
                rows[b], sems[b])

        def start_wb(k, b):
            pltpu.async_copy(
                rows[b], o1_hbm.at[pl.ds((wid + k * NW) * CHUNK, CHUNK)],
                semws[b])

        def wait_wb(k, b):
            pltpu.make_async_copy(
                rows[b], o1_hbm.at[pl.ds((wid + k * NW) * CHUNK, CHUNK)],
                semws[b]).wait()

        start_a(0, 0)
        for k in range(CPW):
            b = k % 2

            @pl.when(k < n_my)
            def _(k=k, b=b):
                pltpu.make_async_copy(a_hbm.at[pl.ds(0, CHUNK)],
                                      rows[b], sems[b]).wait()
                if k + 1 < CPW:
                    @pl.when(k + 1 < n_my)
                    def _():
                        if k >= 1:
                            wait_wb(k - 1, 1 - b)
                        start_a(k + 1, 1 - b)
                pltpu.sync_copy(
                    b_hbm.at[idx_all.at[pl.ds((2 * k + 1) * CHUNK, CHUNK)]],
                    rows[b], add=True)
                start_wb(k, b)

        for k in range(CPW - 3, CPW):
            @pl.when((k == n_my - 1) | (k == n_my - 2))
            def _(k=k):
                wait_wb(k, k % 2)

    return functools.partial(
        pl.kernel,
        out_type=jax.ShapeDtypeStruct((n_edges, D), jnp.float32),
        mesh=_mesh,
        scratch_types=[
            pltpu.VMEM((2 * CPW * CHUNK,), jnp.int32),
            pltpu.VMEM((CHUNK, D), jnp.float32),
            pltpu.VMEM((CHUNK, D), jnp.float32),
            pltpu.SemaphoreType.DMA,
            pltpu.SemaphoreType.DMA,
            pltpu.SemaphoreType.DMA,
            pltpu.SemaphoreType.DMA,
        ],
    )(body)


def _permute_idx(src, dst):
    pad = NW * CPW * CHUNK - EH
    s = jnp.concatenate([src, jnp.zeros((pad,), jnp.int32)])
    d = jnp.concatenate([dst, jnp.zeros((pad,), jnp.int32)])
    s = s.reshape(CPW, NW, CHUNK).transpose(1, 0, 2)
    d = d.reshape(CPW, NW, CHUNK).transpose(1, 0, 2)
    inter = jnp.stack([s, d], axis=2)
    return inter.reshape(NW, 2 * CPW * CHUNK)


_gather_half = _make_gather(EH)


def _scatter_body(ea_hbm, eb_hbm, idxpa_hbm, idxpb_hbm, zero_hbm, out_hbm,
                  idx_all, rows0, rows1, sem0, sem1, sema0, sema1, shared):
    c = lax.axis_index("c")
    s = lax.axis_index("s")
    wid = s * NC + c
    row0 = s * ROWS_PER_TILE
    tail0 = NS * ROWS_PER_TILE
    rows = (rows0, rows1)
    sems = (sem0, sem1)
    semas = (sema0, sema1)
    pltpu.sync_copy(zero_hbm.at[pl.ds(row0, ROWS_PER_TILE)],
                    shared.at[pl.ds(row0, ROWS_PER_TILE)])

    @pl.when(s == NS - 1)
    def _():
        pltpu.sync_copy(zero_hbm.at[pl.ds(tail0, ROWS_TAIL)],
                        shared.at[pl.ds(tail0, ROWS_TAIL)])

    plsc.subcore_barrier()

    nchunks = EH // CHUNK
    n_my = (nchunks - wid + NW - 1) // NW
    for e_hbm, idxp_hbm in ((ea_hbm, idxpa_hbm), (eb_hbm, idxpb_hbm)):
        pltpu.sync_copy(idxp_hbm.at[wid], idx_all)

        def start_load(k, b, e_hbm=e_hbm):
            pltpu.async_copy(e_hbm.at[pl.ds((wid + k * NW) * CHUNK, CHUNK)],
                             rows[b], sems[b])

        def start_add(k, b):
            pltpu.async_copy(
                rows[b],
                shared.at[idx_all.at[pl.ds((2 * k + 1) * CHUNK, CHUNK)]],
                semas[b], add=True)

        def wait_add(k, b):
            pltpu.make_async_copy(
                rows[b],
                shared.at[idx_all.at[pl.ds((2 * k + 1) * CHUNK, CHUNK)]],
                semas[b]).wait()

        start_load(0, 0)
        for k in range(CPW):
            b = k % 2

            @pl.when(k < n_my)
            def _(k=k, b=b, e_hbm=e_hbm, start_load=start_load,
                  start_add=start_add, wait_add=wait_add):
                pltpu.make_async_copy(e_hbm.at[pl.ds(0, CHUNK)],
                                      rows[b], sems[b]).wait()
                if k + 1 < CPW:
                    @pl.when(k + 1 < n_my)
                    def _():
                        if k >= 1:
                            wait_add(k - 1, 1 - b)
                        start_load(k + 1, 1 - b)
                start_add(k, b)

        for k in range(CPW - 3, CPW):
            @pl.when((k == n_my - 1) | (k == n_my - 2))
            def _(k=k, wait_add=wait_add):
                wait_add(k, k % 2)

    plsc.subcore_barrier()
    pltpu.sync_copy(shared.at[pl.ds(row0, ROWS_PER_TILE)],
                    out_hbm.at[c, pl.ds(row0, ROWS_PER_TILE)])

    @pl.when(s == NS - 1)
    def _():
        pltpu.sync_copy(shared.at[pl.ds(tail0, ROWS_TAIL)],
                        out_hbm.at[c, pl.ds(tail0, ROWS_TAIL)])


_scatter_call = functools.partial(
    pl.kernel,
    out_type=jax.ShapeDtypeStruct((NC, N_NODES, D), jnp.float32),
    mesh=_mesh,
    scratch_types=[
        pltpu.VMEM((2 * CPW * CHUNK,), jnp.int32),
        pltpu.VMEM((CHUNK, D), jnp.float32),
        pltpu.VMEM((CHUNK, D), jnp.float32),
        pltpu.SemaphoreType.DMA,
        pltpu.SemaphoreType.DMA,
        pltpu.SemaphoreType.DMA,
        pltpu.SemaphoreType.DMA,
        pltpu.VMEM_SHARED((N_NODES, D), jnp.float32),
    ],
)(_scatter_body)


BE = 2000
BN = 2000


def _proj_body(hn, ws, wd, out_a, out_b):
    x = hn[...]
    out_a[...] = jnp.dot(x, ws[...], preferred_element_type=jnp.float32)
    out_b[...] = jnp.dot(x, wd[...], preferred_element_type=jnp.float32)


def _edge_body(g12, he, w1, b1, w2, b2, g, bb, out):
    he_v = he[...]
    x = g12[...] + b1[...] + jnp.dot(
        he_v, w1[...], preferred_element_type=jnp.float32)
    h = jnp.maximum(x, 0.0)
    y = jnp.dot(h, w2[...], preferred_element_type=jnp.float32) + b2[...]
    mu = jnp.mean(y, axis=-1, keepdims=True)
    yc = y - mu
    var = jnp.mean(yc * yc, axis=-1, keepdims=True)
    out[...] = he_v + yc * lax.rsqrt(var + 1e-5) * g[...] + bb[...]


def _node_body(hn, p0, p1, w1a, w1b, b1, w2, b2, g, bb, ws, wd,
               out_h, out_a, out_b):
    hn_v = hn[...]
    agg = p0[...] + p1[...]
    x = (jnp.dot(hn_v, w1a[...], preferred_element_type=jnp.float32)
         + jnp.dot(agg, w1b[...], preferred_element_type=jnp.float32)
         + b1[...])
    h = jnp.maximum(x, 0.0)
    y = jnp.dot(h, w2[...], preferred_element_type=jnp.float32) + b2[...]
    mu = jnp.mean(y, axis=-1, keepdims=True)
    yc = y - mu
    var = jnp.mean(yc * yc, axis=-1, keepdims=True)
    hn_new = hn_v + yc * lax.rsqrt(var + 1e-5) * g[...] + bb[...]
    out_h[...] = hn_new
    out_a[...] = jnp.dot(hn_new, ws[...], preferred_element_type=jnp.float32)
    out_b[...] = jnp.dot(hn_new, wd[...], preferred_element_type=jnp.float32)


def _row_spec(bs):
    return pl.BlockSpec((bs, D), lambda i: (i, 0))


def _w_spec():
    return pl.BlockSpec((D, D), lambda i: (0, 0))


def _b_spec():
    return pl.BlockSpec((1, D), lambda i: (0, 0))


_proj_call = pl.pallas_call(
    _proj_body,
    grid=(N_NODES // BN,),
    in_specs=[_row_spec(BN), _w_spec(), _w_spec()],
    out_specs=[_row_spec(BN), _row_spec(BN)],
    out_shape=[jax.ShapeDtypeStruct((N_NODES, D), jnp.float32)] * 2,
)

_edge_call = pl.pallas_call(
    _edge_body,
    grid=(EH // BE,),
    in_specs=[_row_spec(BE), _row_spec(BE),
              _w_spec(), _b_spec(), _w_spec(), _b_spec(),
              _b_spec(), _b_spec()],
    out_specs=_row_spec(BE),
    out_shape=jax.ShapeDtypeStruct((EH, D), jnp.float32),
)

_node_call = pl.pallas_call(
    _node_body,
    grid=(N_NODES // BN,),
    in_specs=[_row_spec(BN), _row_spec(BN), _row_spec(BN),
              _w_spec(), _w_spec(), _b_spec(), _w_spec(), _b_spec(),
              _b_spec(), _b_spec(), _w_spec(), _w_spec()],
    out_specs=[_row_spec(BN), _row_spec(BN), _row_spec(BN)],
    out_shape=[jax.ShapeDtypeStruct((N_NODES, D), jnp.float32)] * 3,
)


def kernel(h_node, h_edge, edge_index, We1, be1, We2, be2, ge, bbe,
           Wn1, bn1, Wn2, bn2, gn, bbn):
    src_a, src_b = edge_index[0, :EH], edge_index[0, EH:]
    dst_a, dst_b = edge_index[1, :EH], edge_index[1, EH:]
    idxp_a = _permute_idx(src_a, dst_a)
    idxp_b = _permute_idx(src_b, dst_b)
    he_a, he_b = h_edge[:EH], h_edge[EH:]
    zeros = jnp.zeros((N_NODES, D), jnp.float32)
    num_convs = We1.shape[0]

    a_proj, b_proj = _proj_call(h_node, We1[0, :D], We1[0, D:2 * D])
    for i in range(num_convs):
        ew = (We1[i, 2 * D:], be1[i].reshape(1, D), We2[i],
              be2[i].reshape(1, D), ge[i].reshape(1, D), bbe[i].reshape(1, D))
        g_a = _gather_half(a_proj, b_proj, idxp_a)
        g_b = _gather_half(a_proj, b_proj, idxp_b)
        he_a = _edge_call(g_a, he_a, *ew)
        he_b = _edge_call(g_b, he_b, *ew)
        partials = _scatter_call(he_a, he_b, idxp_a, idxp_b, zeros)
        j = min(i + 1, num_convs - 1)
        h_node, a_proj, b_proj = _node_call(
            h_node, partials[0], partials[1],
            Wn1[i, :D], Wn1[i, D:], bn1[i].reshape(1, D),
            Wn2[i], bn2[i].reshape(1, D),
            gn[i].reshape(1, D), bbn[i].reshape(1, D),
            We1[j, :D], We1[j, D:2 * D])
    return h_node, jnp.concatenate([he_a, he_b], axis=0)
